# Initial kernel scaffold; baseline (speedup 1.0000x reference)
import jax
import jax.numpy as jnp
from jax.experimental import pallas as pl


def kernel(inputs, voc, cpr_w, cpr_b, sm_w, sm_b):
    raise NotImplementedError("write your pallas kernel here")


# speedup vs baseline: 2.8653x; 2.8653x over previous
import functools

import jax
import jax.numpy as jnp
from jax import lax
from jax.experimental import pallas as pl
from jax.experimental.pallas import tpu as pltpu
from jax.experimental.pallas import tpu_sc as plsc

_B, _L, _V, _D, _C, _R = 1024, 20, 1000, 128, 128, 7
_S = _B * 2
_NC, _NS = 2, 16
_NW = _NC * _NS
_SEG_W = _S // _NW
_CH = 4
_RPC = _CH * _L
_NCHUNK = _SEG_W // _CH
_LANES = 16


def _sc_segment_sums(idx_flat, voc):
    mesh = plsc.VectorSubcoreMesh(core_axis_name="c", subcore_axis_name="s")

    @functools.partial(
        pl.kernel,
        mesh=mesh,
        out_type=jax.ShapeDtypeStruct((_S * _D,), jnp.float32),
        scratch_types=[
            pltpu.VMEM((_SEG_W * _L,), jnp.int32),
            pltpu.VMEM((_RPC, _D), jnp.float32),
            pltpu.VMEM((_SEG_W * _D,), jnp.float32),
            pltpu.SemaphoreType.DMA,
        ],
    )
    def body(idx_hbm, voc_hbm, out_hbm, idx_v, rows_v, acc_v, sem):
        wid = lax.axis_index("s") * _NC + lax.axis_index("c")
        pltpu.sync_copy(idx_hbm.at[pl.ds(wid * _SEG_W * _L, _SEG_W * _L)], idx_v)

        def chunk(g, carry):
            pltpu.async_copy(
                voc_hbm.at[idx_v.at[pl.ds(g * _RPC, _RPC)]], rows_v, sem
            ).wait()
            for s in range(_CH):
                for j in range(_D // _LANES):
                    acc = rows_v[s * _L, pl.ds(j * _LANES, _LANES)]
                    for r in range(1, _L):
                        acc = acc + rows_v[s * _L + r, pl.ds(j * _LANES, _LANES)]
                    acc_v[pl.ds((g * _CH + s) * _D + j * _LANES, _LANES)] = acc
            return carry

        lax.fori_loop(0, _NCHUNK, chunk, 0)
        pltpu.sync_copy(acc_v, out_hbm.at[pl.ds(wid * _SEG_W * _D, _SEG_W * _D)])

    return body(idx_flat, voc)


def _mlp_body(x_ref, w1_ref, b1_ref, w2_ref, b2_ref, o_ref):
    x = x_ref[...]
    h = jnp.dot(x, w1_ref[...], preferred_element_type=jnp.float32) + b1_ref[...]
    h = jnp.where(h >= 0, h, 0.01 * h)
    logits = jnp.dot(h, w2_ref[...], preferred_element_type=jnp.float32) + b2_ref[...]
    col = lax.broadcasted_iota(jnp.int32, logits.shape, 1)
    masked = jnp.where(col < _R, logits, -jnp.inf)
    mx = jnp.max(masked, axis=1, keepdims=True)
    e = jnp.where(col < _R, jnp.exp(masked - mx), 0.0)
    lse = jnp.log(jnp.sum(e, axis=1, keepdims=True)) + mx
    o_ref[...] = masked - lse


def kernel(inputs, voc, cpr_w, cpr_b, sm_w, sm_b):
    idx_flat = inputs.astype(jnp.int32).reshape(_S * _L)
    sums = _sc_segment_sums(idx_flat, voc)
    x = sums.reshape(_B, 2 * _D)

    w1 = cpr_w.T
    b1 = cpr_b.reshape(1, _C)
    w2 = jnp.zeros((_C, _C), jnp.float32).at[:, :_R].set(sm_w.T)
    b2 = jnp.zeros((1, _C), jnp.float32).at[0, :_R].set(sm_b)

    out_pad = pl.pallas_call(
        _mlp_body,
        out_shape=jax.ShapeDtypeStruct((_B, _C), jnp.float32),
    )(x, w1, b1, w2, b2)
    return out_pad[:, :_R]
